# baseline (speedup 1.0000x reference)
import functools

import jax
import jax.numpy as jnp
from jax import lax
from jax.experimental import pallas as pl
from jax.experimental.pallas import tpu as pltpu
from jax.experimental.pallas import tpu_sc as plsc

TOP_K = 100
ITEM = 100000
D = 64
B = 1024

BLK = 2048
NBLK = 49
NPAD = BLK * NBLK
CHUNK = 32
NCHUNK = NPAD // CHUNK
SUBBLK = BLK // CHUNK

NWORK = 32
ROWS_PER = B // NWORK
HOTCAP = 128
CAP = 256

NEG = -3.0e38
INT_MIN = -2147483648


def _k1_body(x_ref, kt_ref, logits_ref, m_ref):
    j = pl.program_id(0)
    x = x_ref[...]
    kt = kt_ref[...]
    lg = lax.dot_general(x, kt, (((1,), (1,)), ((), ())),
                         preferred_element_type=jnp.float32)
    gid = j * BLK + lax.broadcasted_iota(jnp.int32, (B, BLK), 1)
    lg = jnp.where(gid < ITEM, lg, NEG)
    logits_ref[...] = lg
    for s in range(SUBBLK):
        m_ref[0, :, s:s + 1] = jnp.max(lg[:, s * CHUNK:(s + 1) * CHUNK],
                                       axis=1, keepdims=True)


def _k1(x, ktpad):
    return pl.pallas_call(
        _k1_body,
        grid=(NBLK,),
        in_specs=[pl.BlockSpec((B, D), lambda j: (0, 0)),
                  pl.BlockSpec((BLK, D), lambda j: (j, 0))],
        out_specs=[pl.BlockSpec((B, BLK), lambda j: (0, j)),
                   pl.BlockSpec((1, B, SUBBLK), lambda j: (j, 0, 0))],
        out_shape=[jax.ShapeDtypeStruct((B, NPAD), jnp.float32),
                   jax.ShapeDtypeStruct((NBLK, B, SUBBLK), jnp.float32)],
    )(x, ktpad)


def _k2_body(m_ref, t_ref):
    m = m_ref[...]
    mb = lax.bitcast_convert_type(m, jnp.int32)
    key = jnp.where(mb >= 0, mb, INT_MIN - mb)

    lo = jnp.min(key, axis=1, keepdims=True)
    hi = jnp.max(key, axis=1, keepdims=True) + 1

    def body(_, carry):
        lo, hi = carry
        mid = (lo & hi) + ((lo ^ hi) >> 1)
        cnt = jnp.sum((key >= mid).astype(jnp.int32), axis=1, keepdims=True)
        ok = cnt >= TOP_K
        return jnp.where(ok, mid, lo), jnp.where(ok, hi, mid)

    lo, hi = lax.fori_loop(0, 34, body, (lo, hi))
    tb = jnp.where(lo >= 0, lo, INT_MIN - lo)
    t_ref[...] = lax.bitcast_convert_type(tb, jnp.float32)


def _k2(m):
    return pl.pallas_call(
        _k2_body,
        out_shape=jax.ShapeDtypeStruct((B, 1), jnp.float32),
    )(m)


PCHUNK = 128
NPCHUNK = NPAD // PCHUNK
RBATCH = 16


def _k3(logits2, m, t):
    mesh = plsc.VectorSubcoreMesh(core_axis_name="c", subcore_axis_name="s",
                                  num_cores=2, num_subcores=16)

    @functools.partial(
        pl.kernel,
        out_type=[jax.ShapeDtypeStruct((B * CAP,), jnp.float32),
                  jax.ShapeDtypeStruct((B * CAP,), jnp.int32)],
        mesh=mesh,
        compiler_params=pltpu.CompilerParams(needs_layout_passes=False),
        scratch_types=[
            pltpu.VMEM((RBATCH * NCHUNK,), jnp.float32),
            pltpu.VMEM((ROWS_PER,), jnp.float32),
            pltpu.VMEM((HOTCAP,), jnp.int32),
            pltpu.VMEM((HOTCAP,), jnp.int32),
            pltpu.VMEM((HOTCAP, PCHUNK), jnp.float32),
            pltpu.VMEM((ROWS_PER * CAP,), jnp.float32),
            pltpu.VMEM((ROWS_PER * CAP,), jnp.int32),
            pltpu.SemaphoreType.DMA,
        ],
    )
    def k3(logits_ref, m_ref, t_ref, cv_out, ci_out,
           m_all, t_v, ids_v, pids_v, dest_v, cv_all, ci_all, sem):
        c = lax.axis_index("c")
        s = lax.axis_index("s")
        wid = s * 2 + c
        base = wid * ROWS_PER

        pltpu.sync_copy(t_ref.at[pl.ds(base, ROWS_PER)], t_v)

        lanes = lax.iota(jnp.int32, 16)
        negv = jnp.full((16,), NEG, jnp.float32)
        zerov = jnp.zeros((16,), jnp.int32)

        def fill(i, _):
            cv_all[pl.ds(i * 16, 16)] = negv
            ci_all[pl.ds(i * 16, 16)] = zerov
            return 0
        lax.fori_loop(0, ROWS_PER * CAP // 16, fill, 0)

        def per_row(r, _):
            grow = base + r
            rb = lax.rem(r, RBATCH)
            tvec = plsc.load_gather(t_v, [jnp.full((16,), r, jnp.int32)])

            padv = jnp.full((16,), grow * NPCHUNK, jnp.int32)
            def fill_ids(q, _):
                pids_v[pl.ds(q * 16, 16)] = padv
                return 0
            lax.fori_loop(0, HOTCAP // 16, fill_ids, 0)

            def scan(j, nhot):
                mv = m_all[pl.ds(rb * NCHUNK + j * 16, 16)]
                msk = mv >= tvec
                loc = j * 16 + lanes
                off = jnp.minimum(nhot, HOTCAP - 16)
                plsc.store_compressed(ids_v.at[pl.ds(off, 16)], loc, mask=msk)
                plsc.store_compressed(
                    pids_v.at[pl.ds(off, 16)],
                    grow * NPCHUNK + lax.shift_right_logical(loc, 2),
                    mask=msk)
                return nhot + plsc.all_reduce_population_count(msk)[0]
            nhot = lax.fori_loop(0, NCHUNK // 16, scan, jnp.int32(0))
            nhot = jnp.minimum(nhot, HOTCAP)

            pltpu.async_copy(logits_ref.at[pids_v], dest_v, sem).wait()

            def filt(h, ncand):
                hs = jnp.full((16,), h, jnp.int32)
                locv = plsc.load_gather(ids_v, [hs])
                ebase = locv * CHUNK
                qoff = (locv & 3) * CHUNK
                def sub(v, ncand):
                    vals = plsc.load_gather(dest_v,
                                            [hs, qoff + v * 16 + lanes])
                    iv = ebase + v * 16 + lanes
                    mk = vals >= tvec
                    off = jnp.minimum(ncand, CAP - 16)
                    plsc.store_compressed(
                        cv_all.at[pl.ds(r * CAP + off, 16)], vals, mask=mk)
                    plsc.store_compressed(
                        ci_all.at[pl.ds(r * CAP + off, 16)], iv, mask=mk)
                    return ncand + plsc.all_reduce_population_count(mk)[0]
                return lax.fori_loop(0, CHUNK // 16, sub, ncand)
            lax.fori_loop(0, nhot, filt, jnp.int32(0))
            return 0

        def per_batch(bi, _):
            span = pl.ds((base + bi * RBATCH) * NCHUNK, RBATCH * NCHUNK)
            pltpu.sync_copy(m_ref.at[span], m_all)
            def row_in_batch(rr, _):
                per_row(bi * RBATCH + rr, 0)
                return 0
            lax.fori_loop(0, RBATCH, row_in_batch, 0)
            return 0
        lax.fori_loop(0, ROWS_PER // RBATCH, per_batch, 0)

        pltpu.sync_copy(cv_all, cv_out.at[pl.ds(base * CAP, ROWS_PER * CAP)])
        pltpu.sync_copy(ci_all, ci_out.at[pl.ds(base * CAP, ROWS_PER * CAP)])

    return k3(logits2, m, t)


def _k4_body(cv_ref, ci_ref, out_ref, cvs):
    cvs[...] = cv_ref[...]
    ci = ci_ref[...]
    lane = lax.broadcasted_iota(jnp.int32, (B, 128), 1)

    def body(i, res):
        cv = cvs[...]
        m = jnp.max(cv, axis=1, keepdims=True)
        isel = jnp.min(jnp.where(cv == m, ci, 2 ** 30),
                       axis=1, keepdims=True)
        res = jnp.where(lane == i, isel.astype(jnp.float32), res)
        kill = (cv == m) & (ci == isel)
        cvs[...] = jnp.where(kill, NEG, cv)
        return res

    res = lax.fori_loop(0, TOP_K, body,
                        jnp.zeros((B, 128), jnp.float32))
    out_ref[...] = res[:, :TOP_K]


def _k4(cv, ci):
    return pl.pallas_call(
        _k4_body,
        out_shape=jax.ShapeDtypeStruct((B, TOP_K), jnp.float32),
        scratch_shapes=[pltpu.VMEM((B, CAP), jnp.float32)],
    )(cv, ci)


def kernel(inputs0, inputs1, kernel):
    del inputs1
    ktpad = jnp.pad(kernel, ((0, NPAD - ITEM), (0, 0)))
    logits, m3 = _k1(inputs0, ktpad)
    m = m3.transpose(1, 0, 2).reshape(B, NCHUNK)
    t = _k2(m)
    cv, ci = _k3(logits.reshape(B * NPCHUNK, PCHUNK), m.reshape(-1),
                 t.reshape(B))
    return _k4(cv.reshape(B, CAP), ci.reshape(B, CAP))


# speedup vs baseline: 1.1727x; 1.1727x over previous
import functools

import jax
import jax.numpy as jnp
from jax import lax
from jax.experimental import pallas as pl
from jax.experimental.pallas import tpu as pltpu
from jax.experimental.pallas import tpu_sc as plsc

TOP_K = 100
ITEM = 100000
D = 64
B = 1024

BLK = 2048
NBLK = 49
NPAD = BLK * NBLK
CHUNK = 128
NCHUNK = NPAD // CHUNK
SUBBLK = BLK // CHUNK

NWORK = 32
ROWS_PER = B // NWORK
HOTCAP = 128
CAP = 256

NEG = -3.0e38
INT_MIN = -2147483648


def _k1_body(x_ref, kt_ref, logits_ref, m_ref):
    j = pl.program_id(0)
    x = x_ref[...]
    kt = kt_ref[...]
    lg = lax.dot_general(x, kt, (((1,), (1,)), ((), ())),
                         preferred_element_type=jnp.float32)
    gid = j * BLK + lax.broadcasted_iota(jnp.int32, (B, BLK), 1)
    lg = jnp.where(gid < ITEM, lg, NEG)
    logits_ref[...] = lg
    for s in range(SUBBLK):
        m_ref[0, :, s:s + 1] = jnp.max(lg[:, s * CHUNK:(s + 1) * CHUNK],
                                       axis=1, keepdims=True)


def _k1(x, ktpad):
    return pl.pallas_call(
        _k1_body,
        grid=(NBLK,),
        in_specs=[pl.BlockSpec((B, D), lambda j: (0, 0)),
                  pl.BlockSpec((BLK, D), lambda j: (j, 0))],
        out_specs=[pl.BlockSpec((B, BLK), lambda j: (0, j)),
                   pl.BlockSpec((1, B, SUBBLK), lambda j: (j, 0, 0))],
        out_shape=[jax.ShapeDtypeStruct((B, NPAD), jnp.float32),
                   jax.ShapeDtypeStruct((NBLK, B, SUBBLK), jnp.float32)],
    )(x, ktpad)


def _k2_body(m_ref, t_ref):
    m = m_ref[...]
    mb = lax.bitcast_convert_type(m, jnp.int32)
    key = jnp.where(mb >= 0, mb, INT_MIN - mb)

    lo = jnp.min(key, axis=1, keepdims=True)
    hi = jnp.max(key, axis=1, keepdims=True) + 1

    def body(_, carry):
        lo, hi = carry
        mid = (lo & hi) + ((lo ^ hi) >> 1)
        cnt = jnp.sum((key >= mid).astype(jnp.int32), axis=1, keepdims=True)
        ok = cnt >= TOP_K
        return jnp.where(ok, mid, lo), jnp.where(ok, hi, mid)

    lo, hi = lax.fori_loop(0, 34, body, (lo, hi))
    tb = jnp.where(lo >= 0, lo, INT_MIN - lo)
    t_ref[...] = lax.bitcast_convert_type(tb, jnp.float32)


def _k2(m):
    return pl.pallas_call(
        _k2_body,
        out_shape=jax.ShapeDtypeStruct((B, 1), jnp.float32),
    )(m)


def _k3(logits2, m, t):
    mesh = plsc.VectorSubcoreMesh(core_axis_name="c", subcore_axis_name="s",
                                  num_cores=2, num_subcores=16)

    @functools.partial(
        pl.kernel,
        out_type=[jax.ShapeDtypeStruct((B * CAP,), jnp.float32),
                  jax.ShapeDtypeStruct((B * CAP,), jnp.int32)],
        mesh=mesh,
        compiler_params=pltpu.CompilerParams(needs_layout_passes=False),
        scratch_types=[
            pltpu.VMEM((ROWS_PER * NCHUNK,), jnp.float32),
            pltpu.VMEM((ROWS_PER,), jnp.float32),
            pltpu.VMEM((HOTCAP,), jnp.int32),
            pltpu.VMEM((HOTCAP, CHUNK), jnp.float32),
            pltpu.VMEM((ROWS_PER * CAP,), jnp.float32),
            pltpu.VMEM((ROWS_PER * CAP,), jnp.int32),
            pltpu.SemaphoreType.DMA,
        ],
    )
    def k3(logits_ref, m_ref, t_ref, cv_out, ci_out,
           m_all, t_v, ids_v, dest_v, cv_all, ci_all, sem):
        c = lax.axis_index("c")
        s = lax.axis_index("s")
        wid = s * 2 + c
        base = wid * ROWS_PER

        pltpu.sync_copy(t_ref.at[pl.ds(base, ROWS_PER)], t_v)
        span = pl.ds(base * NCHUNK, ROWS_PER * NCHUNK)
        pltpu.sync_copy(m_ref.at[span], m_all)

        lanes = lax.iota(jnp.int32, 16)
        negv = jnp.full((16,), NEG, jnp.float32)
        zerov = jnp.zeros((16,), jnp.int32)

        def fill(i, _):
            cv_all[pl.ds(i * 16, 16)] = negv
            ci_all[pl.ds(i * 16, 16)] = zerov
            return 0
        lax.fori_loop(0, ROWS_PER * CAP // 16, fill, 0)
        def per_row(r, _):
            grow = base + r
            tvec = plsc.load_gather(t_v, [jnp.full((16,), r, jnp.int32)])

            padv = jnp.full((16,), grow * NCHUNK, jnp.int32)
            def fill_ids(q, _):
                ids_v[pl.ds(q * 16, 16)] = padv
                return 0
            lax.fori_loop(0, HOTCAP // 16, fill_ids, 0)

            def scan(j, nhot):
                mv = m_all[pl.ds(r * NCHUNK + j * 16, 16)]
                msk = mv >= tvec
                cid = grow * NCHUNK + j * 16 + lanes
                off = jnp.minimum(nhot, HOTCAP - 16)
                plsc.store_compressed(ids_v.at[pl.ds(off, 16)], cid, mask=msk)
                return nhot + plsc.all_reduce_population_count(msk)[0]
            nhot = lax.fori_loop(0, NCHUNK // 16, scan, jnp.int32(0))
            nhot = jnp.minimum(nhot, HOTCAP)

            pltpu.async_copy(logits_ref.at[ids_v], dest_v, sem).wait()

            def filt(h, ncand):
                gidv = plsc.load_gather(ids_v, [jnp.full((16,), h, jnp.int32)])
                ebase = (gidv - grow * NCHUNK) * CHUNK
                def sub(v, ncand):
                    vals = dest_v.at[h][pl.ds(v * 16, 16)]
                    iv = ebase + v * 16 + lanes
                    mk = vals >= tvec
                    off = jnp.minimum(ncand, CAP - 16)
                    plsc.store_compressed(
                        cv_all.at[pl.ds(r * CAP + off, 16)], vals, mask=mk)
                    plsc.store_compressed(
                        ci_all.at[pl.ds(r * CAP + off, 16)], iv, mask=mk)
                    return ncand + plsc.all_reduce_population_count(mk)[0]
                return lax.fori_loop(0, CHUNK // 16, sub, ncand)
            lax.fori_loop(0, nhot, filt, jnp.int32(0))
            return 0

        lax.fori_loop(0, ROWS_PER, per_row, 0)

        pltpu.sync_copy(cv_all, cv_out.at[pl.ds(base * CAP, ROWS_PER * CAP)])
        pltpu.sync_copy(ci_all, ci_out.at[pl.ds(base * CAP, ROWS_PER * CAP)])

    return k3(logits2, m, t)


def _k4_body(cv_ref, ci_ref, out_ref, cvs):
    cvs[...] = cv_ref[...]
    ci = ci_ref[...]
    lane = lax.broadcasted_iota(jnp.int32, (B, 128), 1)

    def body(i, res):
        cv = cvs[...]
        m = jnp.max(cv, axis=1, keepdims=True)
        isel = jnp.min(jnp.where(cv == m, ci, 2 ** 30),
                       axis=1, keepdims=True)
        res = jnp.where(lane == i, isel.astype(jnp.float32), res)
        kill = (cv == m) & (ci == isel)
        cvs[...] = jnp.where(kill, NEG, cv)
        return res

    res = lax.fori_loop(0, TOP_K, body,
                        jnp.zeros((B, 128), jnp.float32))
    out_ref[...] = res[:, :TOP_K]


def _k4(cv, ci):
    return pl.pallas_call(
        _k4_body,
        out_shape=jax.ShapeDtypeStruct((B, TOP_K), jnp.float32),
        scratch_shapes=[pltpu.VMEM((B, CAP), jnp.float32)],
    )(cv, ci)


def kernel(inputs0, inputs1, kernel):
    del inputs1
    ktpad = jnp.pad(kernel, ((0, NPAD - ITEM), (0, 0)))
    logits, m3 = _k1(inputs0, ktpad)
    m = m3.transpose(1, 0, 2).reshape(B, NCHUNK)
    t = _k2(m)
    cv, ci = _k3(logits.reshape(B * NCHUNK, CHUNK), m.reshape(-1), t.reshape(B))
    return _k4(cv.reshape(B, CAP), ci.reshape(B, CAP))
